# baseline (speedup 1.0000x reference)
import jax
import jax.numpy as jnp
from jax import lax
from jax.experimental import pallas as pl
from jax.experimental.pallas import tpu as pltpu
from jax.experimental.pallas import tpu_sc as plsc

NUM_HEADS = 16
NUM_BUCKETS = 32
QLEN = 2048
KLEN = 2048
TQ = 2048
WWIN = 2 * KLEN
HALF = WWIN // 2
LANES = 16

_LARGE_THRESHOLDS = (19, 21, 24, 27, 31, 35, 40, 46, 52, 59, 67, 77, 87, 99, 113)


def _sc_lookup_body(table_t_hbm, out_hbm, trow, wbuf):
    h = lax.axis_index("s")
    pltpu.sync_copy(table_t_hbm.at[pl.ds(h * NUM_BUCKETS, NUM_BUCKETS)], trow)
    tlo = trow[pl.ds(0, LANES)]
    thi = trow[pl.ds(LANES, LANES)]
    max_exact = NUM_BUCKETS // 2

    def gather(vec, idx):
        return jnp.take_along_axis(vec, idx, axis=0, mode="promise_in_bounds")

    def body_pos(it, carry):
        j = it * LANES + lax.iota(jnp.int32, LANES)
        idx = jnp.minimum(KLEN - j, LANES - 1)
        wbuf[pl.ds(it * LANES, LANES)] = gather(thi, idx)
        return carry

    def body_neg(it, carry):
        d = it * LANES + lax.iota(jnp.int32, LANES)
        s = jnp.zeros((LANES,), jnp.int32)
        for t in _LARGE_THRESHOLDS:
            s = s + jnp.minimum(jnp.maximum(d - (t - 1), 0), 1)
        lo = gather(tlo, jnp.minimum(d, LANES - 1))
        hi = gather(thi, s)
        small = jnp.logical_and(1 <= d, d < max_exact)
        wbuf[pl.ds(HALF + it * LANES, LANES)] = jnp.where(small, lo, hi)
        return carry

    lax.fori_loop(0, HALF // LANES, body_pos, 0, unroll=8)
    lax.fori_loop(0, HALF // LANES, body_neg, 0, unroll=8)

    pltpu.sync_copy(wbuf, out_hbm.at[pl.ds(h * WWIN, WWIN)])


def _expand_kernel(w_ref, out_ref):
    wrow = w_ref[0, 0, :][None, :]
    ch = 512
    for c in range(TQ // ch):
        w = jnp.broadcast_to(wrow, (ch, WWIN))
        rolled = pltpu.roll(w, c * ch, 1, stride=1, stride_axis=0)
        out_ref[0, c * ch:(c + 1) * ch, :] = rolled[:, KLEN:]


def kernel(table, qlen, klen):
    table_t = table.T.reshape(-1)
    w_sc = pl.kernel(
        _sc_lookup_body,
        out_type=jax.ShapeDtypeStruct((NUM_HEADS * WWIN,), jnp.float32),
        mesh=plsc.VectorSubcoreMesh(
            core_axis_name="c", subcore_axis_name="s", num_cores=1),
        scratch_types=[
            pltpu.VMEM((NUM_BUCKETS,), jnp.float32),
            pltpu.VMEM((WWIN,), jnp.float32),
        ],
    )(table_t)
    w_all = w_sc.reshape(NUM_HEADS, 1, WWIN)
    out = pl.pallas_call(
        _expand_kernel,
        grid=(NUM_HEADS,),
        in_specs=[pl.BlockSpec((1, 1, WWIN), lambda h: (h, 0, 0))],
        out_specs=pl.BlockSpec((1, TQ, KLEN), lambda h: (h, 0, 0)),
        out_shape=jax.ShapeDtypeStruct((NUM_HEADS, QLEN, KLEN), jnp.float32),
        compiler_params=pltpu.CompilerParams(
            dimension_semantics=("parallel",),
        ),
    )(w_all)
    return out
